# baseline (speedup 1.0000x reference)
import functools

import jax
import jax.numpy as jnp
from jax import lax
from jax.experimental import pallas as pl
from jax.experimental.pallas import tpu as pltpu
from jax.experimental.pallas import tpu_sc as plsc

Q_EXP = 0.3
K_TRUNC = 0.5
BATCH_N = 4096
CLASSES_N = 1000
TRAIN_N = 50000

NUM_CORES = 2
NUM_SUBCORES = 16
NUM_TILES = NUM_CORES * NUM_SUBCORES
PER_TILE = BATCH_N // NUM_TILES
CONST_TERM = (1.0 - K_TRUNC ** Q_EXP) / Q_EXP

COLS_BLK = 2048
GRID_N = BATCH_N // COLS_BLK


def _wgather_body(index_h, weight_f, out_h, idx_v, w_v, sem):
    wid = lax.axis_index("s") * NUM_CORES + lax.axis_index("c")
    base = wid * PER_TILE
    pltpu.sync_copy(index_h.at[pl.ds(base, PER_TILE)], idx_v)
    pltpu.async_copy(weight_f.at[idx_v], w_v, sem).wait()
    pltpu.sync_copy(w_v, out_h.at[pl.ds(base, PER_TILE)])


_sc_wgather = functools.partial(
    pl.kernel,
    out_type=jax.ShapeDtypeStruct((BATCH_N,), jnp.float32),
    mesh=plsc.VectorSubcoreMesh(
        core_axis_name="c", subcore_axis_name="s",
        num_cores=NUM_CORES, num_subcores=NUM_SUBCORES,
    ),
    scratch_types=[
        pltpu.VMEM((PER_TILE,), jnp.int32),
        pltpu.VMEM((PER_TILE,), jnp.float32),
        pltpu.SemaphoreType.DMA,
    ],
)(_wgather_body)


def _loss_body(lt_ref, tgt_ref, w_ref, out_ref):
    t = tgt_ref[0, 0, :]
    wv = w_ref[0, 0, :]
    rows = lax.broadcasted_iota(jnp.int32, (CLASSES_N, COLS_BLK), 0)
    yg = jnp.sum(jnp.where(rows == t[None, :], lt_ref[...], 0.0), axis=0)
    g = (1.0 - yg ** Q_EXP) * (1.0 / Q_EXP) - CONST_TERM
    part = jnp.sum(g * wv) * (1.0 / BATCH_N)

    @pl.when(pl.program_id(0) == 0)
    def _():
        out_ref[...] = jnp.zeros_like(out_ref)

    out_ref[...] += part.reshape(1, 1)


_tc_loss = pl.pallas_call(
    _loss_body,
    grid=(GRID_N,),
    in_specs=[
        pl.BlockSpec((CLASSES_N, COLS_BLK), lambda i: (0, i)),
        pl.BlockSpec((1, 1, COLS_BLK), lambda i: (i, 0, 0)),
        pl.BlockSpec((1, 1, COLS_BLK), lambda i: (i, 0, 0)),
    ],
    out_specs=pl.BlockSpec((1, 1), lambda i: (0, 0)),
    out_shape=jax.ShapeDtypeStruct((1, 1), jnp.float32),
)


def kernel(logits, targets, index, weight):
    w = _sc_wgather(index, weight.reshape(-1))
    out = _tc_loss(
        pltpu.with_memory_space_constraint(logits.T, pltpu.MemorySpace.HBM),
        targets.reshape(GRID_N, 1, COLS_BLK),
        w.reshape(GRID_N, 1, COLS_BLK),
    )
    return out[0, 0]


# speedup vs baseline: 1.0194x; 1.0194x over previous
import functools

import jax
import jax.numpy as jnp
from jax import lax
from jax.experimental import pallas as pl
from jax.experimental.pallas import tpu as pltpu
from jax.experimental.pallas import tpu_sc as plsc

Q_EXP = 0.3
K_TRUNC = 0.5
BATCH_N = 4096
CLASSES_N = 1000
TRAIN_N = 50000

NUM_CORES = 2
NUM_SUBCORES = 16
NUM_TILES = NUM_CORES * NUM_SUBCORES
PER_TILE = BATCH_N // NUM_TILES
CONST_TERM = (1.0 - K_TRUNC ** Q_EXP) / Q_EXP

COLS_BLK = 1024
GRID_N = BATCH_N // COLS_BLK


def _wgather_body(index_h, weight_f, out_h, idx_v, w_v, sem):
    wid = lax.axis_index("s") * NUM_CORES + lax.axis_index("c")
    base = wid * PER_TILE
    pltpu.sync_copy(index_h.at[pl.ds(base, PER_TILE)], idx_v)
    pltpu.async_copy(weight_f.at[idx_v], w_v, sem).wait()
    pltpu.sync_copy(w_v, out_h.at[pl.ds(base, PER_TILE)])


_sc_wgather = functools.partial(
    pl.kernel,
    out_type=jax.ShapeDtypeStruct((BATCH_N,), jnp.float32),
    mesh=plsc.VectorSubcoreMesh(
        core_axis_name="c", subcore_axis_name="s",
        num_cores=NUM_CORES, num_subcores=NUM_SUBCORES,
    ),
    scratch_types=[
        pltpu.VMEM((PER_TILE,), jnp.int32),
        pltpu.VMEM((PER_TILE,), jnp.float32),
        pltpu.SemaphoreType.DMA,
    ],
)(_wgather_body)


def _loss_body(lt_ref, tgt_ref, w_ref, out_ref):
    t = tgt_ref[0, 0, :]
    wv = w_ref[0, 0, :]
    rows = lax.broadcasted_iota(jnp.int32, (CLASSES_N, COLS_BLK), 0)
    yg = jnp.sum(jnp.where(rows == t[None, :], lt_ref[...], 0.0), axis=0)
    g = (1.0 - yg ** Q_EXP) * (1.0 / Q_EXP) - CONST_TERM
    part = jnp.sum(g * wv) * (1.0 / BATCH_N)

    @pl.when(pl.program_id(0) == 0)
    def _():
        out_ref[...] = jnp.zeros_like(out_ref)

    out_ref[...] += part.reshape(1, 1)


_tc_loss = pl.pallas_call(
    _loss_body,
    grid=(GRID_N,),
    in_specs=[
        pl.BlockSpec((CLASSES_N, COLS_BLK), lambda i: (0, i)),
        pl.BlockSpec((1, 1, COLS_BLK), lambda i: (i, 0, 0)),
        pl.BlockSpec((1, 1, COLS_BLK), lambda i: (i, 0, 0)),
    ],
    out_specs=pl.BlockSpec((1, 1), lambda i: (0, 0)),
    out_shape=jax.ShapeDtypeStruct((1, 1), jnp.float32),
)


def kernel(logits, targets, index, weight):
    w = _sc_wgather(index, weight.reshape(-1))
    out = _tc_loss(
        pltpu.with_memory_space_constraint(logits.T, pltpu.MemorySpace.HBM),
        targets.reshape(GRID_N, 1, COLS_BLK),
        w.reshape(GRID_N, 1, COLS_BLK),
    )
    return out[0, 0]


# speedup vs baseline: 1.0541x; 1.0340x over previous
import functools

import jax
import jax.numpy as jnp
from jax import lax
from jax.experimental import pallas as pl
from jax.experimental.pallas import tpu as pltpu
from jax.experimental.pallas import tpu_sc as plsc

Q_EXP = 0.3
K_TRUNC = 0.5
BATCH_N = 4096
CLASSES_N = 1000
TRAIN_N = 50000

NUM_CORES = 1
NUM_SUBCORES = 16
NUM_TILES = NUM_CORES * NUM_SUBCORES
PER_TILE = BATCH_N // NUM_TILES
CONST_TERM = (1.0 - K_TRUNC ** Q_EXP) / Q_EXP

COLS_BLK = 1024
GRID_N = BATCH_N // COLS_BLK


def _wgather_body(index_h, weight_f, out_h, idx_v, w_v, sem):
    wid = lax.axis_index("s") * NUM_CORES + lax.axis_index("c")
    base = wid * PER_TILE
    pltpu.sync_copy(index_h.at[pl.ds(base, PER_TILE)], idx_v)
    pltpu.async_copy(weight_f.at[idx_v], w_v, sem).wait()
    pltpu.sync_copy(w_v, out_h.at[pl.ds(base, PER_TILE)])


_sc_wgather = functools.partial(
    pl.kernel,
    out_type=jax.ShapeDtypeStruct((BATCH_N,), jnp.float32),
    mesh=plsc.VectorSubcoreMesh(
        core_axis_name="c", subcore_axis_name="s",
        num_cores=NUM_CORES, num_subcores=NUM_SUBCORES,
    ),
    scratch_types=[
        pltpu.VMEM((PER_TILE,), jnp.int32),
        pltpu.VMEM((PER_TILE,), jnp.float32),
        pltpu.SemaphoreType.DMA,
    ],
)(_wgather_body)


def _loss_body(lt_ref, tgt_ref, w_ref, out_ref):
    t = tgt_ref[0, 0, :]
    wv = w_ref[0, 0, :]
    rows = lax.broadcasted_iota(jnp.int32, (CLASSES_N, COLS_BLK), 0)
    yg = jnp.sum(jnp.where(rows == t[None, :], lt_ref[...], 0.0), axis=0)
    g = (1.0 - yg ** Q_EXP) * (1.0 / Q_EXP) - CONST_TERM
    part = jnp.sum(g * wv) * (1.0 / BATCH_N)

    @pl.when(pl.program_id(0) == 0)
    def _():
        out_ref[...] = jnp.zeros_like(out_ref)

    out_ref[...] += part.reshape(1, 1)


_tc_loss = pl.pallas_call(
    _loss_body,
    grid=(GRID_N,),
    in_specs=[
        pl.BlockSpec((CLASSES_N, COLS_BLK), lambda i: (0, i)),
        pl.BlockSpec((1, 1, COLS_BLK), lambda i: (i, 0, 0)),
        pl.BlockSpec((1, 1, COLS_BLK), lambda i: (i, 0, 0)),
    ],
    out_specs=pl.BlockSpec((1, 1), lambda i: (0, 0)),
    out_shape=jax.ShapeDtypeStruct((1, 1), jnp.float32),
)


def kernel(logits, targets, index, weight):
    w = _sc_wgather(index, weight.reshape(-1))
    out = _tc_loss(
        pltpu.with_memory_space_constraint(logits.T, pltpu.MemorySpace.HBM),
        targets.reshape(GRID_N, 1, COLS_BLK),
        w.reshape(GRID_N, 1, COLS_BLK),
    )
    return out[0, 0]
